# baseline (speedup 1.0000x reference)
import functools
import jax
import jax.numpy as jnp
from jax import lax
from jax.experimental import pallas as pl
from jax.experimental.pallas import tpu as pltpu
from jax.experimental.pallas import tpu_sc as plsc

NS = 7
NR = 128
CUT = 6.0

_NC = 2
_NSC = 16
_NW = _NC * _NSC


def _sc_mesh():
    return plsc.VectorSubcoreMesh(core_axis_name="c", subcore_axis_name="s",
                                  num_cores=_NC, num_subcores=_NSC)


def _chunk(per_w, cap=128):
    c = min(per_w, cap)
    c -= c % 8
    while c >= 8:
        if per_w % c == 0:
            return c
        c -= 8
    raise ValueError(per_w)


def _sc_gather(table, idx):
    M = idx.shape[0]
    D = table.shape[1]
    per_w = M // _NW
    CH = _chunk(per_w)
    n = per_w // CH

    @functools.partial(
        pl.kernel, mesh=_sc_mesh(),
        out_type=jax.ShapeDtypeStruct((M, D), jnp.float32),
        scratch_types=[pltpu.VMEM((CH,), jnp.int32),
                       pltpu.VMEM((CH,), jnp.int32),
                       pltpu.VMEM((CH, D), jnp.float32),
                       pltpu.VMEM((CH, D), jnp.float32),
                       pltpu.SemaphoreType.DMA,
                       pltpu.SemaphoreType.DMA],
    )
    def k(table_h, idx_h, out_h, i0, i1, r0, r1, s0, s1):
        wid = lax.axis_index("s") * _NC + lax.axis_index("c")
        base = wid * per_w

        def gath(c, ibuf, buf, sem):
            pltpu.sync_copy(idx_h.at[pl.ds(base + c * CH, CH)], ibuf)
            pltpu.async_copy(table_h.at[ibuf], buf, sem)

        def wait(ibuf, buf, sem):
            pltpu.make_async_copy(table_h.at[ibuf], buf, sem).wait()

        def store(c, buf):
            pltpu.sync_copy(buf, out_h.at[pl.ds(base + c * CH, CH)])

        gath(0, i0, r0, s0)

        def pair(i, _):
            e = 2 * i
            gath(e + 1, i1, r1, s1)
            wait(i0, r0, s0)
            store(e, r0)

            @pl.when(e + 2 < n)
            def _():
                gath(e + 2, i0, r0, s0)
            wait(i1, r1, s1)
            store(e + 1, r1)
            return 0

        lax.fori_loop(0, n // 2, pair, 0)
        if n % 2:
            wait(i0, r0, s0)
            store(n - 1, r0)

    return k(table, idx)


def _sc_gather_small(table, idx):
    M = idx.shape[0]
    R, D = table.shape
    per_w = M // _NW
    CH = _chunk(per_w)
    n = per_w // CH
    rpt = (R // _NSC) & ~7
    tail = R - _NSC * rpt

    @functools.partial(
        pl.kernel, mesh=_sc_mesh(),
        out_type=jax.ShapeDtypeStruct((M, D), jnp.float32),
        scratch_types=[pltpu.VMEM((CH,), jnp.int32),
                       pltpu.VMEM((CH,), jnp.int32),
                       pltpu.VMEM((CH, D), jnp.float32),
                       pltpu.VMEM((CH, D), jnp.float32),
                       pltpu.VMEM_SHARED((R, D), jnp.float32),
                       pltpu.SemaphoreType.DMA,
                       pltpu.SemaphoreType.DMA],
    )
    def k(table_h, idx_h, out_h, i0, i1, r0, r1, tb, s0, s1):
        sid = lax.axis_index("s")
        wid = sid * _NC + lax.axis_index("c")
        base = wid * per_w
        pltpu.sync_copy(table_h.at[pl.ds(sid * rpt, rpt)],
                        tb.at[pl.ds(sid * rpt, rpt)])
        if tail:
            @pl.when(sid == _NSC - 1)
            def _():
                pltpu.sync_copy(table_h.at[pl.ds(_NSC * rpt, tail)],
                                tb.at[pl.ds(_NSC * rpt, tail)])
        plsc.subcore_barrier()

        def gath(c, ibuf, buf, sem):
            pltpu.sync_copy(idx_h.at[pl.ds(base + c * CH, CH)], ibuf)
            pltpu.async_copy(tb.at[ibuf], buf, sem)

        def wait(ibuf, buf, sem):
            pltpu.make_async_copy(tb.at[ibuf], buf, sem).wait()

        def store(c, buf):
            pltpu.sync_copy(buf, out_h.at[pl.ds(base + c * CH, CH)])

        gath(0, i0, r0, s0)

        def pair(i, _):
            e = 2 * i
            gath(e + 1, i1, r1, s1)
            wait(i0, r0, s0)
            store(e, r0)

            @pl.when(e + 2 < n)
            def _():
                gath(e + 2, i0, r0, s0)
            wait(i1, r1, s1)
            store(e + 1, r1)
            return 0

        lax.fori_loop(0, n // 2, pair, 0)
        if n % 2:
            wait(i0, r0, s0)
            store(n - 1, r0)

    return k(table, idx)


def _zero_fill(zv, ZR, D):
    z16 = jnp.zeros((16,), jnp.float32)

    def zrow(r, _):
        for c in range(D // 16):
            zv[r, pl.ds(c * 16, 16)] = z16
        return 0
    lax.fori_loop(0, ZR, zrow, 0)


def _sc_segsum(vals_list, idx, nseg):
    K = len(vals_list)
    M, D = vals_list[0].shape
    per_w = M // _NW
    CH = _chunk(per_w)
    n = per_w // CH
    rpt = (nseg // _NSC) & ~7
    tail = nseg - _NSC * rpt
    assert tail % 8 == 0
    ZR = next(c for c in range(min(rpt, 128), 0, -1)
              if rpt % c == 0 and tail % c == 0)
    nz = rpt // ZR

    @functools.partial(
        pl.kernel, mesh=_sc_mesh(),
        out_type=jax.ShapeDtypeStruct((K, _NC, nseg, D), jnp.float32),
        scratch_types=[pltpu.VMEM((CH,), jnp.int32),
                       pltpu.VMEM((CH,), jnp.int32),
                       pltpu.VMEM((CH, D), jnp.float32),
                       pltpu.VMEM((CH, D), jnp.float32),
                       pltpu.VMEM((ZR, D), jnp.float32),
                       pltpu.VMEM_SHARED((nseg, D), jnp.float32),
                       pltpu.SemaphoreType.DMA,
                       pltpu.SemaphoreType.DMA,
                       pltpu.SemaphoreType.DMA,
                       pltpu.SemaphoreType.DMA],
    )
    def k(*refs):
        vals_h = refs[:K]
        idx_h = refs[K]
        out_h = refs[K + 1]
        i0, i1, r0, r1, zv, acc, sa0, sa1, st0, st1 = refs[K + 2:]
        cid = lax.axis_index("c")
        sid = lax.axis_index("s")
        wid = sid * _NC + cid
        base = wid * per_w
        _zero_fill(zv, ZR, D)
        for kk in range(K):
            def zcp(z, _):
                pltpu.sync_copy(zv, acc.at[pl.ds(sid * rpt + z * ZR, ZR)])
                return 0
            lax.fori_loop(0, nz, zcp, 0)

            @pl.when(sid == _NSC - 1)
            def _():
                def zct(z, _):
                    pltpu.sync_copy(zv, acc.at[pl.ds(_NSC * rpt + z * ZR, ZR)])
                    return 0
                lax.fori_loop(0, tail // ZR, zct, 0)
            plsc.subcore_barrier()

            def issue(g, ib, rb, sa, st):
                off = base + g * CH
                pltpu.async_copy(idx_h.at[pl.ds(off, CH)], ib, sa)
                pltpu.async_copy(vals_h[kk].at[pl.ds(off, CH)], rb, st)

            def finish(g, ib, rb, sa, st):
                off = base + g * CH
                pltpu.make_async_copy(idx_h.at[pl.ds(off, CH)], ib, sa).wait()
                pltpu.make_async_copy(vals_h[kk].at[pl.ds(off, CH)], rb,
                                      st).wait()
                pltpu.sync_copy(rb, acc.at[ib], add=True)

            issue(0, i0, r0, sa0, st0)

            def body(i, _):
                e = 2 * i
                issue(e + 1, i1, r1, sa1, st1)
                finish(e, i0, r0, sa0, st0)

                @pl.when(e + 2 < n)
                def _():
                    issue(e + 2, i0, r0, sa0, st0)
                finish(e + 1, i1, r1, sa1, st1)
                return 0
            lax.fori_loop(0, n // 2, body, 0)
            if n % 2:
                finish(n - 1, i0, r0, sa0, st0)
            plsc.subcore_barrier()
            pltpu.sync_copy(acc.at[pl.ds(sid * rpt, rpt)],
                            out_h.at[kk, cid, pl.ds(sid * rpt, rpt)])

            @pl.when(sid == _NSC - 1)
            def _():
                pltpu.sync_copy(acc.at[pl.ds(_NSC * rpt, tail)],
                                out_h.at[kk, cid, pl.ds(_NSC * rpt, tail)])
            plsc.subcore_barrier()

    return k(*vals_list, idx)


_TRI_R = 12800
_TRI_DUMMY = 384


def _zero_fill_bf(zv, ZR, D):
    z216 = jnp.zeros((2, 16), jnp.bfloat16)

    def zrow(r, _):
        for c in range(D // 16):
            zv[pl.ds(2 * r, 2), pl.ds(c * 16, 16)] = z216
        return 0
    lax.fori_loop(0, ZR // 2, zrow, 0)


def _sc_tri_dense(t, ca_px, E):
    Tp, D = t.shape
    per_w = Tp // _NW
    CHT = 64
    nch = per_w // CHT
    R = _TRI_R if E >= _TRI_R else E
    nrng = (E + R - 1) // R
    AR = R + _TRI_DUMMY
    rpt_z = AR // _NSC
    nzf = rpt_z // CHT
    rem = rpt_z - nzf * CHT

    @functools.partial(
        pl.kernel, mesh=_sc_mesh(),
        out_type=jax.ShapeDtypeStruct((_NC, E, D), jnp.float32),
        scratch_types=[pltpu.VMEM((CHT,), jnp.int32),
                       pltpu.VMEM((CHT,), jnp.int32),
                       pltpu.VMEM((CHT,), jnp.int32),
                       pltpu.VMEM((CHT, D), jnp.float32),
                       pltpu.VMEM((CHT, D), jnp.float32),
                       pltpu.VMEM((CHT, D), jnp.float32),
                       pltpu.SemaphoreType.DMA,
                       pltpu.SemaphoreType.DMA,
                       pltpu.SemaphoreType.DMA,
                       pltpu.SemaphoreType.DMA,
                       pltpu.VMEM_SHARED((AR, D), jnp.float32)],
    )
    def k(t_h, ca_h, out_h, cav0, cav1, locv, zv, r0, r1,
          sa0, sa1, st0, st1, acc):
        cid = lax.axis_index("c")
        sid = lax.axis_index("s")
        wid = sid * _NC + cid
        base = wid * per_w
        lane = lax.broadcasted_iota(jnp.int32, (16,), 0)
        _zero_fill(zv, 128, D)

        for r in range(nrng):
            lo = r * R
            Rr = R if lo + R <= E else E - lo

            def zcp(z, _):
                pltpu.sync_copy(zv, acc.at[pl.ds(sid * rpt_z + z * CHT, CHT)])
                return 0
            lax.fori_loop(0, nzf, zcp, 0)
            if rem:
                pltpu.sync_copy(zv.at[pl.ds(0, rem)],
                                acc.at[pl.ds(sid * rpt_z + nzf * CHT, rem)])
            plsc.subcore_barrier()

            dummy = R + sid * 8 + (lane & 7)

            def issue(jc, cav, rows, sa, st):
                pltpu.async_copy(ca_h.at[pl.ds(base + jc * CHT, CHT)], cav, sa)
                pltpu.async_copy(t_h.at[pl.ds(base + jc * CHT, CHT)], rows, st)

            def finish(jc, cav, rows, sa, st):
                pltpu.make_async_copy(ca_h.at[pl.ds(base + jc * CHT, CHT)],
                                      cav, sa).wait()
                pltpu.make_async_copy(t_h.at[pl.ds(base + jc * CHT, CHT)],
                                      rows, st).wait()
                for v in range(CHT // 16):
                    lvec = cav[pl.ds(v * 16, 16)] - lo
                    loc = jnp.where(lvec >= 0,
                                    jnp.where(lvec < Rr, lvec, dummy), dummy)
                    locv[pl.ds(v * 16, 16)] = loc
                pltpu.sync_copy(rows, acc.at[locv], add=True)

            issue(0, cav0, r0, sa0, st0)

            def pair(i, _):
                e = 2 * i
                issue(e + 1, cav1, r1, sa1, st1)
                finish(e, cav0, r0, sa0, st0)

                @pl.when(e + 2 < nch)
                def _():
                    issue(e + 2, cav0, r0, sa0, st0)
                finish(e + 1, cav1, r1, sa1, st1)
                return 0
            lax.fori_loop(0, nch // 2, pair, 0)
            if nch % 2:
                finish(nch - 1, cav0, r0, sa0, st0)
            plsc.subcore_barrier()
            rpt_o = (Rr // _NSC) & ~7
            tl = Rr - _NSC * rpt_o
            pltpu.sync_copy(acc.at[pl.ds(sid * rpt_o, rpt_o)],
                            out_h.at[cid, pl.ds(lo + sid * rpt_o, rpt_o)])
            if tl:
                @pl.when(sid == _NSC - 1)
                def _():
                    pltpu.sync_copy(
                        acc.at[pl.ds(_NSC * rpt_o, tl)],
                        out_h.at[cid, pl.ds(lo + _NSC * rpt_o, tl)])
            plsc.subcore_barrier()

    return k(t, ca_px)


def _swish(x):
    return x / (1.0 + jnp.exp(-x))


def _blk(n, cap):
    b = min(n, cap)
    while b > 8:
        if n % b == 0 and b % 8 == 0:
            return b
        b -= 8
    return n


def _full(a):
    return pl.BlockSpec(a.shape, lambda i: (0,) * a.ndim)


def _rows(shape, bs):
    return pl.BlockSpec((bs,) + shape[1:], lambda i: (i,) + (0,) * (len(shape) - 1))


def _edge_body(hs_r, hd_r, ps, pd, W_edge, b_edge, W_rbf_h, W_rbf_out,
               W_cbf_r, Wd0, m_o, sh_o, so_o, vb_o, x_o):
    hs = hs_r[...]
    hd = hd_r[...]
    vec = pd[:, 0:16] - ps[:, 0:16]
    d = jnp.sqrt(jnp.sum(vec * vec, axis=-1, keepdims=True) + 1e-9)
    V = vec / d
    x = d / CUT
    offs = jax.lax.broadcasted_iota(jnp.int32, (1, NR), 1).astype(jnp.float32) / (NR - 1)
    coeff = -0.5 / (1.0 / (NR - 1)) ** 2
    g = jnp.exp(coeff * (x - offs) ** 2)
    p5 = x * x
    p5 = p5 * p5 * x
    poly = 1.0 + (-21.0) * p5 + 35.0 * (p5 * x) + (-15.0) * (p5 * x * x)
    env = jnp.where(x < 1.0, poly, 0.0)
    rbf = g * env
    We = W_edge[...]
    pre = (jnp.dot(hs, We[0:128], preferred_element_type=jnp.float32)
           + jnp.dot(hd, We[128:256], preferred_element_type=jnp.float32)
           + jnp.dot(rbf, We[256:384], preferred_element_type=jnp.float32)
           + b_edge[...])
    m = _swish(pre)
    m_o[...] = m
    x = _swish(jnp.dot(m, Wd0[...], preferred_element_type=jnp.float32))
    x_o[...] = jnp.concatenate([x, jnp.zeros_like(x)], axis=-1)
    sh_o[...] = jnp.dot(rbf, W_rbf_h[...], preferred_element_type=jnp.float32)
    so_o[...] = jnp.dot(rbf, W_rbf_out[...], preferred_element_type=jnp.float32)
    rw = jnp.dot(rbf, W_cbf_r[...], preferred_element_type=jnp.float32)
    z = jnp.zeros_like(rw)
    vb_o[...] = jnp.concatenate([V, rw, z, z, z, z, z, z], axis=-1)


def _edge_stage(hs0, hd0, psrc, pdst, W_edge, b_edge, W_rbf_h, W_rbf_out,
                W_cbf_r, Wd0):
    E = hs0.shape[0]
    bs = _blk(E, 2000)
    grid = (E // bs,)
    outs = (
        jax.ShapeDtypeStruct((E, 128), jnp.float32),
        jax.ShapeDtypeStruct((E, 128), jnp.float32),
        jax.ShapeDtypeStruct((E, 128), jnp.float32),
        jax.ShapeDtypeStruct((E, 128), jnp.float32),
        jax.ShapeDtypeStruct((E, 128), jnp.float32),
    )
    return pl.pallas_call(
        _edge_body,
        grid=grid,
        in_specs=[_rows((E, 128), bs)] * 4 + [
                  _full(W_edge), _full(b_edge), _full(W_rbf_h),
                  _full(W_rbf_out), _full(W_cbf_r), _full(Wd0)],
        out_specs=tuple(_rows(o.shape, bs) for o in outs),
        out_shape=outs,
    )(hs0, hd0, psrc, pdst, W_edge, b_edge, W_rbf_h, W_rbf_out, W_cbf_r, Wd0)


def _basis_body(va, vc, W_cbf_s, out):
    cos = jnp.clip(jnp.sum(va[:, 0:16] * vc[:, 0:16], axis=-1, keepdims=True),
                   -1.0, 1.0)
    ps = [jnp.ones_like(cos), cos]
    for l in range(2, NS):
        ps.append(((2 * l - 1) * cos * ps[-1] - (l - 1) * ps[-2]) / l)
    cbf = jnp.concatenate(ps[:NS], axis=-1)
    out[...] = (jnp.dot(cbf, W_cbf_s[...], preferred_element_type=jnp.float32)
                * vc[:, 16:32])


def _basis_stage(va, vc, W_cbf_s):
    T = va.shape[0]
    bs = _blk(T, 4000)
    return pl.pallas_call(
        _basis_body,
        grid=(T // bs,),
        in_specs=[_rows(va.shape, bs), _rows(vc.shape, bs), _full(W_cbf_s)],
        out_specs=_rows((T, 16), bs),
        out_shape=jax.ShapeDtypeStruct((T, 16), jnp.float32),
    )(va, vc, W_cbf_s)


def _x_body(m, Wd, out):
    x = _swish(jnp.dot(m[...], Wd[...], preferred_element_type=jnp.float32))
    out[...] = jnp.concatenate([x, jnp.zeros_like(x)], axis=-1)


def _x_stage(m, Wd):
    E = m.shape[0]
    bs = _blk(E, 2000)
    return pl.pallas_call(
        _x_body,
        grid=(E // bs,),
        in_specs=[_rows(m.shape, bs), _full(Wd)],
        out_specs=_rows((E, 128), bs),
        out_shape=jax.ShapeDtypeStruct((E, 128), jnp.float32),
    )(m, Wd)


def _t_body(xg, basis, Wb2t, Wbil, out):
    bas = jnp.dot(basis[...], Wb2t[...], preferred_element_type=jnp.float32)
    t = xg[:, 0:64] * bas
    t = _swish(jnp.dot(t, Wbil[...], preferred_element_type=jnp.float32))
    out[...] = jnp.concatenate([t, jnp.zeros_like(t)], axis=-1)


def _t_stage(xg, basis, Wb2t, Wbil):
    T = xg.shape[0]
    bs = _blk(T, 4000)
    return pl.pallas_call(
        _t_body,
        grid=(T // bs,),
        in_specs=[_rows(xg.shape, bs), _rows(basis.shape, bs),
                  _full(Wb2t), _full(Wbil)],
        out_specs=_rows((T, 128), bs),
        out_shape=jax.ShapeDtypeStruct((T, 128), jnp.float32),
    )(xg, basis, Wb2t, Wbil)


def _m1_body(agg, m, sh, Wup, Wres, m_o, msh_o):
    a = agg[0, :, 0:64] + agg[1, :, 0:64]
    m1 = m[...] + _swish(jnp.dot(a, Wup[...], preferred_element_type=jnp.float32))
    m2 = m1 + _swish(jnp.dot(m1, Wres[...], preferred_element_type=jnp.float32))
    m_o[...] = m2
    msh_o[...] = m2 * sh[...]


def _m1_stage(agg, m, sh, Wup, Wres):
    E = m.shape[0]
    bs = _blk(E, 2000)
    outs = (jax.ShapeDtypeStruct((E, 128), jnp.float32),
            jax.ShapeDtypeStruct((E, 128), jnp.float32))
    aggspec = pl.BlockSpec((2, bs, agg.shape[2]), lambda i: (0, i, 0))
    return pl.pallas_call(
        _m1_body,
        grid=(E // bs,),
        in_specs=[aggspec, _rows(m.shape, bs), _rows(sh.shape, bs),
                  _full(Wup), _full(Wres)],
        out_specs=tuple(_rows(o.shape, bs) for o in outs),
        out_shape=outs,
    )(agg, m, sh, Wup, Wres)


def _h_body(amsg, h, Watom, We2, h_o, hs_o, hd_o):
    a = amsg[0] + amsg[1]
    hn = h[...] + _swish(jnp.dot(a, Watom[...], preferred_element_type=jnp.float32))
    W = We2[...]
    h_o[...] = hn
    hs_o[...] = jnp.dot(hn, W[0:128], preferred_element_type=jnp.float32)
    hd_o[...] = jnp.dot(hn, W[128:256], preferred_element_type=jnp.float32)


def _h_stage(amsg, h, Watom, We2):
    N = h.shape[0]
    bs = _blk(N, 2000)
    outs = (jax.ShapeDtypeStruct((N, 128), jnp.float32),) * 3
    return pl.pallas_call(
        _h_body,
        grid=(N // bs,),
        in_specs=[pl.BlockSpec((2, bs, 128), lambda i: (0, i, 0)),
                  _rows(h.shape, bs), _full(Watom), _full(We2)],
        out_specs=tuple(_rows(o.shape, bs) for o in outs),
        out_shape=outs,
    )(amsg, h, Watom, We2)


def _m2_body_x(m, ghs, ghd, We2, Wd, out, x_o):
    W = We2[...]
    pre = ghs[...] + ghd[...] + jnp.dot(m[...], W[256:384],
                                        preferred_element_type=jnp.float32)
    mn = m[...] + _swish(pre)
    out[...] = mn
    x = _swish(jnp.dot(mn, Wd[...], preferred_element_type=jnp.float32))
    x_o[...] = jnp.concatenate([x, jnp.zeros_like(x)], axis=-1)


def _m2_body(m, ghs, ghd, We2, out):
    W = We2[...]
    pre = ghs[...] + ghd[...] + jnp.dot(m[...], W[256:384],
                                        preferred_element_type=jnp.float32)
    out[...] = m[...] + _swish(pre)


def _m2_stage(m, ghs, ghd, We2, Wd=None):
    E = m.shape[0]
    bs = _blk(E, 2000)
    if Wd is None:
        return pl.pallas_call(
            _m2_body,
            grid=(E // bs,),
            in_specs=[_rows(m.shape, bs), _rows(ghs.shape, bs),
                      _rows(ghd.shape, bs), _full(We2)],
            out_specs=_rows((E, 128), bs),
            out_shape=jax.ShapeDtypeStruct((E, 128), jnp.float32),
        )(m, ghs, ghd, We2)
    outs = (jax.ShapeDtypeStruct((E, 128), jnp.float32),
            jax.ShapeDtypeStruct((E, 128), jnp.float32))
    return pl.pallas_call(
        _m2_body_x,
        grid=(E // bs,),
        in_specs=[_rows(m.shape, bs), _rows(ghs.shape, bs),
                  _rows(ghd.shape, bs), _full(We2), _full(Wd)],
        out_specs=tuple(_rows(o.shape, bs) for o in outs),
        out_shape=outs,
    )(m, ghs, ghd, We2, Wd)


def _out_body(m0, m1, m2, m3, so, V, WF, xm0, xm1, xm2, xm3, fv_o):
    s = so[...]
    W = WF[...]
    xs = []
    fsum = None
    for k, mk in enumerate((m0, m1, m2, m3)):
        xm = mk[...] * s
        xs.append(xm)
        fe = jnp.dot(xm, W[k], preferred_element_type=jnp.float32)
        fsum = fe if fsum is None else fsum + fe
    xm0[...], xm1[...], xm2[...], xm3[...] = xs
    fv_o[...] = fsum * V[...]


def _out_stage(ms, so, V, WF):
    E = so.shape[0]
    bs = _blk(E, 2000)
    outs = tuple(jax.ShapeDtypeStruct((E, 128), jnp.float32) for _ in range(5))
    return pl.pallas_call(
        _out_body,
        grid=(E // bs,),
        in_specs=[_rows((E, 128), bs)] * 4 + [_rows(so.shape, bs),
                  _rows(V.shape, bs), _full(WF)],
        out_specs=tuple(_rows(o.shape, bs) for o in outs),
        out_shape=outs,
    )(*ms, so, V, WF)


def _energy_body(xa, W1, W2, out):
    acc = None
    for k in range(4):
        xk = xa[k, 0] + xa[k, 1]
        e = jnp.dot(_swish(jnp.dot(xk, W1[...][k],
                                   preferred_element_type=jnp.float32)),
                    W2[...][k], preferred_element_type=jnp.float32)
        acc = e if acc is None else acc + e
    out[...] = acc


def _energy_stage(xap, W1, W2):
    N = xap.shape[2]
    bs = _blk(N, 2000)
    return pl.pallas_call(
        _energy_body,
        grid=(N // bs,),
        in_specs=[pl.BlockSpec((4, 2, bs, 128), lambda i: (0, 0, i, 0)),
                  _full(W1), _full(W2)],
        out_specs=_rows((N, 1), bs),
        out_shape=jax.ShapeDtypeStruct((N, 1), jnp.float32),
    )(xap, W1, W2)


def kernel(atomic_numbers, pos, edge_index, id3_ba, id3_ca, atom_table,
           W_edge, b_edge, W_cbf_s, W_cbf_r, W_down, W_b2t, W_bil, W_up,
           W_res, W_rbf_h, W_atom, W_e2, W_rbf_out, W_out1, W_out2, W_F):
    N = pos.shape[0]
    E = edge_index.shape[1]
    T = id3_ba.shape[0]
    src = edge_index[0].astype(jnp.int32)
    dst = edge_index[1].astype(jnp.int32)
    ba = id3_ba.astype(jnp.int32)
    ca = id3_ca.astype(jnp.int32)

    Npad = ((N + _NW * 8 - 1) // (_NW * 8)) * (_NW * 8)
    an_pad = jnp.pad(atomic_numbers.astype(jnp.int32), (0, Npad - N))
    Tp = ((T + _NW * 128 - 1) // (_NW * 128)) * (_NW * 128)
    ba_p = jnp.concatenate([ba, jnp.zeros((Tp - T,), jnp.int32)])
    ca_p = jnp.concatenate([ca, jnp.zeros((Tp - T,), jnp.int32)])
    ca_px = jnp.concatenate([ca, jnp.full((Tp - T,), jnp.int32(1 << 20))])

    h_pad = _sc_gather(atom_table, an_pad)
    h = h_pad[:N]
    pos128 = jnp.pad(pos, ((0, 0), (0, 125)))
    hs0 = _sc_gather_small(h, src)
    hd0 = _sc_gather_small(h, dst)
    psrc = _sc_gather_small(pos128, src)
    pdst = _sc_gather_small(pos128, dst)

    m, scale_h, scale_out, vb, x = _edge_stage(
        hs0, hd0, psrc, pdst, W_edge, b_edge, W_rbf_h, W_rbf_out, W_cbf_r,
        W_down[0])

    va = _sc_gather(vb, ba_p)
    vc = _sc_gather(vb, ca_p)
    basis = _basis_stage(va, vc, W_cbf_s)

    ms = [m]
    for b in range(3):
        xg = _sc_gather(x, ba_p)
        t = _t_stage(xg, basis, W_b2t[b], W_bil[b])
        aggp = _sc_tri_dense(t, ca_px, E)
        m, msh = _m1_stage(aggp, m, scale_h, W_up[b], W_res[b])
        amsgp = _sc_segsum([msh], dst, N)
        h, hsN, hdN = _h_stage(amsgp[0], h, W_atom[b], W_e2[b])
        ghs = _sc_gather_small(hsN, src)
        ghd = _sc_gather_small(hdN, dst)
        if b < 2:
            m, x = _m2_stage(m, ghs, ghd, W_e2[b], W_down[b + 1])
        else:
            m = _m2_stage(m, ghs, ghd, W_e2[b])
        ms.append(m)

    xm0, xm1, xm2, xm3, fV = _out_stage(ms, scale_out, vb, W_F)
    xap = _sc_segsum([xm0, xm1, xm2, xm3], dst, N)
    Eat = _energy_stage(xap, W_out1, W_out2)
    fvp = _sc_segsum([fV], dst, N)
    F = (fvp[0, 0] + fvp[0, 1])[:, :3]
    energy = jnp.sum(Eat).reshape(1)
    return jnp.concatenate([F.reshape(-1), energy])


# speedup vs baseline: 1.0108x; 1.0108x over previous
import functools
import jax
import jax.numpy as jnp
from jax import lax
from jax.experimental import pallas as pl
from jax.experimental.pallas import tpu as pltpu
from jax.experimental.pallas import tpu_sc as plsc

NS = 7
NR = 128
CUT = 6.0

_NC = 2
_NSC = 16
_NW = _NC * _NSC


def _sc_mesh():
    return plsc.VectorSubcoreMesh(core_axis_name="c", subcore_axis_name="s",
                                  num_cores=_NC, num_subcores=_NSC)


def _chunk(per_w, cap=128):
    c = min(per_w, cap)
    c -= c % 8
    while c >= 8:
        if per_w % c == 0:
            return c
        c -= 8
    raise ValueError(per_w)


def _sc_gather(table, idx):
    M = idx.shape[0]
    D = table.shape[1]
    per_w = M // _NW
    CH = _chunk(per_w)
    n = per_w // CH

    @functools.partial(
        pl.kernel, mesh=_sc_mesh(),
        out_type=jax.ShapeDtypeStruct((M, D), jnp.float32),
        scratch_types=[pltpu.VMEM((CH,), jnp.int32),
                       pltpu.VMEM((CH,), jnp.int32),
                       pltpu.VMEM((CH, D), jnp.float32),
                       pltpu.VMEM((CH, D), jnp.float32),
                       pltpu.SemaphoreType.DMA,
                       pltpu.SemaphoreType.DMA],
    )
    def k(table_h, idx_h, out_h, i0, i1, r0, r1, s0, s1):
        wid = lax.axis_index("s") * _NC + lax.axis_index("c")
        base = wid * per_w

        def gath(c, ibuf, buf, sem):
            pltpu.sync_copy(idx_h.at[pl.ds(base + c * CH, CH)], ibuf)
            pltpu.async_copy(table_h.at[ibuf], buf, sem)

        def wait(ibuf, buf, sem):
            pltpu.make_async_copy(table_h.at[ibuf], buf, sem).wait()

        def store(c, buf):
            pltpu.sync_copy(buf, out_h.at[pl.ds(base + c * CH, CH)])

        gath(0, i0, r0, s0)

        def pair(i, _):
            e = 2 * i
            gath(e + 1, i1, r1, s1)
            wait(i0, r0, s0)
            store(e, r0)

            @pl.when(e + 2 < n)
            def _():
                gath(e + 2, i0, r0, s0)
            wait(i1, r1, s1)
            store(e + 1, r1)
            return 0

        lax.fori_loop(0, n // 2, pair, 0)
        if n % 2:
            wait(i0, r0, s0)
            store(n - 1, r0)

    return k(table, idx)


def _sc_gather_small(table, idx):
    M = idx.shape[0]
    R, D = table.shape
    per_w = M // _NW
    CH = _chunk(per_w)
    n = per_w // CH
    rpt = (R // _NSC) & ~7
    tail = R - _NSC * rpt

    @functools.partial(
        pl.kernel, mesh=_sc_mesh(),
        out_type=jax.ShapeDtypeStruct((M, D), jnp.float32),
        scratch_types=[pltpu.VMEM((CH,), jnp.int32),
                       pltpu.VMEM((CH,), jnp.int32),
                       pltpu.VMEM((CH, D), jnp.float32),
                       pltpu.VMEM((CH, D), jnp.float32),
                       pltpu.VMEM_SHARED((R, D), jnp.float32),
                       pltpu.SemaphoreType.DMA,
                       pltpu.SemaphoreType.DMA],
    )
    def k(table_h, idx_h, out_h, i0, i1, r0, r1, tb, s0, s1):
        sid = lax.axis_index("s")
        wid = sid * _NC + lax.axis_index("c")
        base = wid * per_w
        pltpu.sync_copy(table_h.at[pl.ds(sid * rpt, rpt)],
                        tb.at[pl.ds(sid * rpt, rpt)])
        if tail:
            @pl.when(sid == _NSC - 1)
            def _():
                pltpu.sync_copy(table_h.at[pl.ds(_NSC * rpt, tail)],
                                tb.at[pl.ds(_NSC * rpt, tail)])
        plsc.subcore_barrier()

        def gath(c, ibuf, buf, sem):
            pltpu.sync_copy(idx_h.at[pl.ds(base + c * CH, CH)], ibuf)
            pltpu.async_copy(tb.at[ibuf], buf, sem)

        def wait(ibuf, buf, sem):
            pltpu.make_async_copy(tb.at[ibuf], buf, sem).wait()

        def store(c, buf):
            pltpu.sync_copy(buf, out_h.at[pl.ds(base + c * CH, CH)])

        gath(0, i0, r0, s0)

        def pair(i, _):
            e = 2 * i
            gath(e + 1, i1, r1, s1)
            wait(i0, r0, s0)
            store(e, r0)

            @pl.when(e + 2 < n)
            def _():
                gath(e + 2, i0, r0, s0)
            wait(i1, r1, s1)
            store(e + 1, r1)
            return 0

        lax.fori_loop(0, n // 2, pair, 0)
        if n % 2:
            wait(i0, r0, s0)
            store(n - 1, r0)

    return k(table, idx)


def _zero_fill(zv, ZR, D):
    z16 = jnp.zeros((16,), jnp.float32)

    def zrow(r, _):
        for c in range(D // 16):
            zv[r, pl.ds(c * 16, 16)] = z16
        return 0
    lax.fori_loop(0, ZR, zrow, 0)


def _sc_segsum(vals_list, idx, nseg):
    K = len(vals_list)
    M, D = vals_list[0].shape
    per_w = M // _NW
    CH = _chunk(per_w)
    n = per_w // CH
    rpt = (nseg // _NSC) & ~7
    tail = nseg - _NSC * rpt
    assert tail % 8 == 0
    ZR = next(c for c in range(min(rpt, 128), 0, -1)
              if rpt % c == 0 and tail % c == 0)
    nz = rpt // ZR

    @functools.partial(
        pl.kernel, mesh=_sc_mesh(),
        out_type=jax.ShapeDtypeStruct((K, _NC, nseg, D), jnp.float32),
        scratch_types=[pltpu.VMEM((CH,), jnp.int32),
                       pltpu.VMEM((CH,), jnp.int32),
                       pltpu.VMEM((CH, D), jnp.float32),
                       pltpu.VMEM((CH, D), jnp.float32),
                       pltpu.VMEM((ZR, D), jnp.float32),
                       pltpu.VMEM_SHARED((nseg, D), jnp.float32),
                       pltpu.SemaphoreType.DMA,
                       pltpu.SemaphoreType.DMA,
                       pltpu.SemaphoreType.DMA,
                       pltpu.SemaphoreType.DMA],
    )
    def k(*refs):
        vals_h = refs[:K]
        idx_h = refs[K]
        out_h = refs[K + 1]
        i0, i1, r0, r1, zv, acc, sa0, sa1, st0, st1 = refs[K + 2:]
        cid = lax.axis_index("c")
        sid = lax.axis_index("s")
        wid = sid * _NC + cid
        base = wid * per_w
        _zero_fill(zv, ZR, D)
        for kk in range(K):
            def zcp(z, _):
                pltpu.sync_copy(zv, acc.at[pl.ds(sid * rpt + z * ZR, ZR)])
                return 0
            lax.fori_loop(0, nz, zcp, 0)

            @pl.when(sid == _NSC - 1)
            def _():
                def zct(z, _):
                    pltpu.sync_copy(zv, acc.at[pl.ds(_NSC * rpt + z * ZR, ZR)])
                    return 0
                lax.fori_loop(0, tail // ZR, zct, 0)
            plsc.subcore_barrier()

            def issue(g, ib, rb, sa, st):
                off = base + g * CH
                pltpu.async_copy(idx_h.at[pl.ds(off, CH)], ib, sa)
                pltpu.async_copy(vals_h[kk].at[pl.ds(off, CH)], rb, st)

            def finish(g, ib, rb, sa, st):
                off = base + g * CH
                pltpu.make_async_copy(idx_h.at[pl.ds(off, CH)], ib, sa).wait()
                pltpu.make_async_copy(vals_h[kk].at[pl.ds(off, CH)], rb,
                                      st).wait()
                pltpu.sync_copy(rb, acc.at[ib], add=True)

            issue(0, i0, r0, sa0, st0)

            def body(i, _):
                e = 2 * i
                issue(e + 1, i1, r1, sa1, st1)
                finish(e, i0, r0, sa0, st0)

                @pl.when(e + 2 < n)
                def _():
                    issue(e + 2, i0, r0, sa0, st0)
                finish(e + 1, i1, r1, sa1, st1)
                return 0
            lax.fori_loop(0, n // 2, body, 0)
            if n % 2:
                finish(n - 1, i0, r0, sa0, st0)
            plsc.subcore_barrier()
            pltpu.sync_copy(acc.at[pl.ds(sid * rpt, rpt)],
                            out_h.at[kk, cid, pl.ds(sid * rpt, rpt)])

            @pl.when(sid == _NSC - 1)
            def _():
                pltpu.sync_copy(acc.at[pl.ds(_NSC * rpt, tail)],
                                out_h.at[kk, cid, pl.ds(_NSC * rpt, tail)])
            plsc.subcore_barrier()

    return k(*vals_list, idx)


_TRI_R = 12800
_TRI_DUMMY = 384


def _zero_fill_bf(zv, ZR, D):
    z216 = jnp.zeros((2, 16), jnp.bfloat16)

    def zrow(r, _):
        for c in range(D // 16):
            zv[pl.ds(2 * r, 2), pl.ds(c * 16, 16)] = z216
        return 0
    lax.fori_loop(0, ZR // 2, zrow, 0)


def _sc_tri_dense(t, ca_px, E):
    Tp, D = t.shape
    per_w = Tp // _NW
    CHT = 64
    nch = per_w // CHT
    R = _TRI_R if E >= _TRI_R else E
    nrng = (E + R - 1) // R
    AR = R + _TRI_DUMMY
    rpt_z = AR // _NSC
    nzf = rpt_z // CHT
    rem = rpt_z - nzf * CHT

    @functools.partial(
        pl.kernel, mesh=_sc_mesh(),
        out_type=jax.ShapeDtypeStruct((_NC, E, D), jnp.float32),
        scratch_types=[pltpu.VMEM((CHT,), jnp.int32),
                       pltpu.VMEM((CHT,), jnp.int32),
                       pltpu.VMEM((CHT,), jnp.int32),
                       pltpu.VMEM((CHT, D), jnp.float32),
                       pltpu.VMEM((CHT, D), jnp.float32),
                       pltpu.VMEM((CHT, D), jnp.float32),
                       pltpu.SemaphoreType.DMA,
                       pltpu.SemaphoreType.DMA,
                       pltpu.SemaphoreType.DMA,
                       pltpu.SemaphoreType.DMA,
                       pltpu.VMEM_SHARED((AR, D), jnp.float32)],
    )
    def k(t_h, ca_h, out_h, cav0, cav1, locv, zv, r0, r1,
          sa0, sa1, st0, st1, acc):
        cid = lax.axis_index("c")
        sid = lax.axis_index("s")
        wid = sid * _NC + cid
        base = wid * per_w
        lane = lax.broadcasted_iota(jnp.int32, (16,), 0)
        _zero_fill(zv, 128, D)

        for r in range(nrng):
            lo = r * R
            Rr = R if lo + R <= E else E - lo

            def zcp(z, _):
                pltpu.sync_copy(zv, acc.at[pl.ds(sid * rpt_z + z * CHT, CHT)])
                return 0
            lax.fori_loop(0, nzf, zcp, 0)
            if rem:
                pltpu.sync_copy(zv.at[pl.ds(0, rem)],
                                acc.at[pl.ds(sid * rpt_z + nzf * CHT, rem)])
            plsc.subcore_barrier()

            dummy = R + sid * 8 + (lane & 7)

            def issue(jc, cav, rows, sa, st):
                pltpu.async_copy(ca_h.at[pl.ds(base + jc * CHT, CHT)], cav, sa)
                pltpu.async_copy(t_h.at[pl.ds(base + jc * CHT, CHT)], rows, st)

            def finish(jc, cav, rows, sa, st):
                pltpu.make_async_copy(ca_h.at[pl.ds(base + jc * CHT, CHT)],
                                      cav, sa).wait()
                pltpu.make_async_copy(t_h.at[pl.ds(base + jc * CHT, CHT)],
                                      rows, st).wait()
                for v in range(CHT // 16):
                    lvec = cav[pl.ds(v * 16, 16)] - lo
                    loc = jnp.where(lvec >= 0,
                                    jnp.where(lvec < Rr, lvec, dummy), dummy)
                    locv[pl.ds(v * 16, 16)] = loc
                pltpu.sync_copy(rows, acc.at[locv], add=True)

            issue(0, cav0, r0, sa0, st0)

            def pair(i, _):
                e = 2 * i
                issue(e + 1, cav1, r1, sa1, st1)
                finish(e, cav0, r0, sa0, st0)

                @pl.when(e + 2 < nch)
                def _():
                    issue(e + 2, cav0, r0, sa0, st0)
                finish(e + 1, cav1, r1, sa1, st1)
                return 0
            lax.fori_loop(0, nch // 2, pair, 0)
            if nch % 2:
                finish(nch - 1, cav0, r0, sa0, st0)
            plsc.subcore_barrier()
            rpt_o = (Rr // _NSC) & ~7
            tl = Rr - _NSC * rpt_o
            pltpu.sync_copy(acc.at[pl.ds(sid * rpt_o, rpt_o)],
                            out_h.at[cid, pl.ds(lo + sid * rpt_o, rpt_o)])
            if tl:
                @pl.when(sid == _NSC - 1)
                def _():
                    pltpu.sync_copy(
                        acc.at[pl.ds(_NSC * rpt_o, tl)],
                        out_h.at[cid, pl.ds(lo + _NSC * rpt_o, tl)])
            plsc.subcore_barrier()

    return k(t, ca_px)


def _swish(x):
    return x / (1.0 + jnp.exp(-x))


def _blk(n, cap):
    b = min(n, cap)
    while b > 8:
        if n % b == 0 and b % 8 == 0:
            return b
        b -= 8
    return n


def _full(a):
    return pl.BlockSpec(a.shape, lambda i: (0,) * a.ndim)


def _rows(shape, bs):
    return pl.BlockSpec((bs,) + shape[1:], lambda i: (i,) + (0,) * (len(shape) - 1))


def _edge_body(gs, gd, W_edge, b_edge, W_rbf_h, W_rbf_out, W_cbf_r, Wd0,
               m_o, sh_o, so_o, vb_o, x_o):
    hs = gs[:, 0:128]
    hd = gd[:, 0:128]
    vec = gd[:, 128:144] - gs[:, 128:144]
    d = jnp.sqrt(jnp.sum(vec * vec, axis=-1, keepdims=True) + 1e-9)
    V = vec / d
    x = d / CUT
    offs = jax.lax.broadcasted_iota(jnp.int32, (1, NR), 1).astype(jnp.float32) / (NR - 1)
    coeff = -0.5 / (1.0 / (NR - 1)) ** 2
    g = jnp.exp(coeff * (x - offs) ** 2)
    p5 = x * x
    p5 = p5 * p5 * x
    poly = 1.0 + (-21.0) * p5 + 35.0 * (p5 * x) + (-15.0) * (p5 * x * x)
    env = jnp.where(x < 1.0, poly, 0.0)
    rbf = g * env
    We = W_edge[...]
    pre = (jnp.dot(hs, We[0:128], preferred_element_type=jnp.float32)
           + jnp.dot(hd, We[128:256], preferred_element_type=jnp.float32)
           + jnp.dot(rbf, We[256:384], preferred_element_type=jnp.float32)
           + b_edge[...])
    m = _swish(pre)
    m_o[...] = m
    x = _swish(jnp.dot(m, Wd0[...], preferred_element_type=jnp.float32))
    x_o[...] = jnp.concatenate([x, jnp.zeros_like(x)], axis=-1)
    sh_o[...] = jnp.dot(rbf, W_rbf_h[...], preferred_element_type=jnp.float32)
    so_o[...] = jnp.dot(rbf, W_rbf_out[...], preferred_element_type=jnp.float32)
    rw = jnp.dot(rbf, W_cbf_r[...], preferred_element_type=jnp.float32)
    z = jnp.zeros_like(rw)
    vb_o[...] = jnp.concatenate([V, rw, z, z, z, z, z, z], axis=-1)


def _edge_stage(gsrc, gdst, W_edge, b_edge, W_rbf_h, W_rbf_out, W_cbf_r, Wd0):
    E = gsrc.shape[0]
    bs = _blk(E, 2000)
    grid = (E // bs,)
    outs = (
        jax.ShapeDtypeStruct((E, 128), jnp.float32),
        jax.ShapeDtypeStruct((E, 128), jnp.float32),
        jax.ShapeDtypeStruct((E, 128), jnp.float32),
        jax.ShapeDtypeStruct((E, 128), jnp.float32),
        jax.ShapeDtypeStruct((E, 128), jnp.float32),
    )
    return pl.pallas_call(
        _edge_body,
        grid=grid,
        in_specs=[_rows(gsrc.shape, bs), _rows(gdst.shape, bs),
                  _full(W_edge), _full(b_edge), _full(W_rbf_h),
                  _full(W_rbf_out), _full(W_cbf_r), _full(Wd0)],
        out_specs=tuple(_rows(o.shape, bs) for o in outs),
        out_shape=outs,
    )(gsrc, gdst, W_edge, b_edge, W_rbf_h, W_rbf_out, W_cbf_r, Wd0)


def _basis_body(va, vc, W_cbf_s, out):
    cos = jnp.clip(jnp.sum(va[:, 0:16] * vc[:, 0:16], axis=-1, keepdims=True),
                   -1.0, 1.0)
    ps = [jnp.ones_like(cos), cos]
    for l in range(2, NS):
        ps.append(((2 * l - 1) * cos * ps[-1] - (l - 1) * ps[-2]) / l)
    cbf = jnp.concatenate(ps[:NS], axis=-1)
    out[...] = (jnp.dot(cbf, W_cbf_s[...], preferred_element_type=jnp.float32)
                * vc[:, 16:32])


def _basis_stage(va, vc, W_cbf_s):
    T = va.shape[0]
    bs = _blk(T, 4000)
    return pl.pallas_call(
        _basis_body,
        grid=(T // bs,),
        in_specs=[_rows(va.shape, bs), _rows(vc.shape, bs), _full(W_cbf_s)],
        out_specs=_rows((T, 16), bs),
        out_shape=jax.ShapeDtypeStruct((T, 16), jnp.float32),
    )(va, vc, W_cbf_s)


def _x_body(m, Wd, out):
    x = _swish(jnp.dot(m[...], Wd[...], preferred_element_type=jnp.float32))
    out[...] = jnp.concatenate([x, jnp.zeros_like(x)], axis=-1)


def _x_stage(m, Wd):
    E = m.shape[0]
    bs = _blk(E, 2000)
    return pl.pallas_call(
        _x_body,
        grid=(E // bs,),
        in_specs=[_rows(m.shape, bs), _full(Wd)],
        out_specs=_rows((E, 128), bs),
        out_shape=jax.ShapeDtypeStruct((E, 128), jnp.float32),
    )(m, Wd)


def _t_body(xg, basis, Wb2t, Wbil, out):
    bas = jnp.dot(basis[...], Wb2t[...], preferred_element_type=jnp.float32)
    t = xg[:, 0:64] * bas
    t = _swish(jnp.dot(t, Wbil[...], preferred_element_type=jnp.float32))
    out[...] = jnp.concatenate([t, jnp.zeros_like(t)], axis=-1)


def _t_stage(xg, basis, Wb2t, Wbil):
    T = xg.shape[0]
    bs = _blk(T, 4000)
    return pl.pallas_call(
        _t_body,
        grid=(T // bs,),
        in_specs=[_rows(xg.shape, bs), _rows(basis.shape, bs),
                  _full(Wb2t), _full(Wbil)],
        out_specs=_rows((T, 128), bs),
        out_shape=jax.ShapeDtypeStruct((T, 128), jnp.float32),
    )(xg, basis, Wb2t, Wbil)


def _m1_body(agg, m, sh, Wup, Wres, m_o, msh_o):
    a = agg[0, :, 0:64] + agg[1, :, 0:64]
    m1 = m[...] + _swish(jnp.dot(a, Wup[...], preferred_element_type=jnp.float32))
    m2 = m1 + _swish(jnp.dot(m1, Wres[...], preferred_element_type=jnp.float32))
    m_o[...] = m2
    msh_o[...] = m2 * sh[...]


def _m1_stage(agg, m, sh, Wup, Wres):
    E = m.shape[0]
    bs = _blk(E, 2000)
    outs = (jax.ShapeDtypeStruct((E, 128), jnp.float32),
            jax.ShapeDtypeStruct((E, 128), jnp.float32))
    aggspec = pl.BlockSpec((2, bs, agg.shape[2]), lambda i: (0, i, 0))
    return pl.pallas_call(
        _m1_body,
        grid=(E // bs,),
        in_specs=[aggspec, _rows(m.shape, bs), _rows(sh.shape, bs),
                  _full(Wup), _full(Wres)],
        out_specs=tuple(_rows(o.shape, bs) for o in outs),
        out_shape=outs,
    )(agg, m, sh, Wup, Wres)


def _h_body(amsg, h, Watom, We2, h_o, hs_o, hd_o):
    a = amsg[0] + amsg[1]
    hn = h[...] + _swish(jnp.dot(a, Watom[...], preferred_element_type=jnp.float32))
    W = We2[...]
    h_o[...] = hn
    hs_o[...] = jnp.dot(hn, W[0:128], preferred_element_type=jnp.float32)
    hd_o[...] = jnp.dot(hn, W[128:256], preferred_element_type=jnp.float32)


def _h_stage(amsg, h, Watom, We2):
    N = h.shape[0]
    bs = _blk(N, 2000)
    outs = (jax.ShapeDtypeStruct((N, 128), jnp.float32),) * 3
    return pl.pallas_call(
        _h_body,
        grid=(N // bs,),
        in_specs=[pl.BlockSpec((2, bs, 128), lambda i: (0, i, 0)),
                  _rows(h.shape, bs), _full(Watom), _full(We2)],
        out_specs=tuple(_rows(o.shape, bs) for o in outs),
        out_shape=outs,
    )(amsg, h, Watom, We2)


def _m2_body_x(m, ghs, ghd, We2, Wd, out, x_o):
    W = We2[...]
    pre = ghs[...] + ghd[...] + jnp.dot(m[...], W[256:384],
                                        preferred_element_type=jnp.float32)
    mn = m[...] + _swish(pre)
    out[...] = mn
    x = _swish(jnp.dot(mn, Wd[...], preferred_element_type=jnp.float32))
    x_o[...] = jnp.concatenate([x, jnp.zeros_like(x)], axis=-1)


def _m2_body(m, ghs, ghd, We2, out):
    W = We2[...]
    pre = ghs[...] + ghd[...] + jnp.dot(m[...], W[256:384],
                                        preferred_element_type=jnp.float32)
    out[...] = m[...] + _swish(pre)


def _m2_stage(m, ghs, ghd, We2, Wd=None):
    E = m.shape[0]
    bs = _blk(E, 2000)
    if Wd is None:
        return pl.pallas_call(
            _m2_body,
            grid=(E // bs,),
            in_specs=[_rows(m.shape, bs), _rows(ghs.shape, bs),
                      _rows(ghd.shape, bs), _full(We2)],
            out_specs=_rows((E, 128), bs),
            out_shape=jax.ShapeDtypeStruct((E, 128), jnp.float32),
        )(m, ghs, ghd, We2)
    outs = (jax.ShapeDtypeStruct((E, 128), jnp.float32),
            jax.ShapeDtypeStruct((E, 128), jnp.float32))
    return pl.pallas_call(
        _m2_body_x,
        grid=(E // bs,),
        in_specs=[_rows(m.shape, bs), _rows(ghs.shape, bs),
                  _rows(ghd.shape, bs), _full(We2), _full(Wd)],
        out_specs=tuple(_rows(o.shape, bs) for o in outs),
        out_shape=outs,
    )(m, ghs, ghd, We2, Wd)


def _out_body(m0, m1, m2, m3, so, V, WF, xm0, xm1, xm2, xm3, fv_o):
    s = so[...]
    W = WF[...]
    xs = []
    fsum = None
    for k, mk in enumerate((m0, m1, m2, m3)):
        xm = mk[...] * s
        xs.append(xm)
        fe = jnp.dot(xm, W[k], preferred_element_type=jnp.float32)
        fsum = fe if fsum is None else fsum + fe
    xm0[...], xm1[...], xm2[...], xm3[...] = xs
    fv_o[...] = fsum * V[...]


def _out_stage(ms, so, V, WF):
    E = so.shape[0]
    bs = _blk(E, 2000)
    outs = tuple(jax.ShapeDtypeStruct((E, 128), jnp.float32) for _ in range(5))
    return pl.pallas_call(
        _out_body,
        grid=(E // bs,),
        in_specs=[_rows((E, 128), bs)] * 4 + [_rows(so.shape, bs),
                  _rows(V.shape, bs), _full(WF)],
        out_specs=tuple(_rows(o.shape, bs) for o in outs),
        out_shape=outs,
    )(*ms, so, V, WF)


def _energy_body(xa, W1, W2, out):
    acc = None
    for k in range(4):
        xk = xa[k, 0] + xa[k, 1]
        e = jnp.dot(_swish(jnp.dot(xk, W1[...][k],
                                   preferred_element_type=jnp.float32)),
                    W2[...][k], preferred_element_type=jnp.float32)
        acc = e if acc is None else acc + e
    out[...] = acc


def _energy_stage(xap, W1, W2):
    N = xap.shape[2]
    bs = _blk(N, 2000)
    return pl.pallas_call(
        _energy_body,
        grid=(N // bs,),
        in_specs=[pl.BlockSpec((4, 2, bs, 128), lambda i: (0, 0, i, 0)),
                  _full(W1), _full(W2)],
        out_specs=_rows((N, 1), bs),
        out_shape=jax.ShapeDtypeStruct((N, 1), jnp.float32),
    )(xap, W1, W2)


def kernel(atomic_numbers, pos, edge_index, id3_ba, id3_ca, atom_table,
           W_edge, b_edge, W_cbf_s, W_cbf_r, W_down, W_b2t, W_bil, W_up,
           W_res, W_rbf_h, W_atom, W_e2, W_rbf_out, W_out1, W_out2, W_F):
    N = pos.shape[0]
    E = edge_index.shape[1]
    T = id3_ba.shape[0]
    src = edge_index[0].astype(jnp.int32)
    dst = edge_index[1].astype(jnp.int32)
    ba = id3_ba.astype(jnp.int32)
    ca = id3_ca.astype(jnp.int32)

    Npad = ((N + _NW * 8 - 1) // (_NW * 8)) * (_NW * 8)
    an_pad = jnp.pad(atomic_numbers.astype(jnp.int32), (0, Npad - N))
    Tp = ((T + _NW * 128 - 1) // (_NW * 128)) * (_NW * 128)
    ba_p = jnp.concatenate([ba, jnp.zeros((Tp - T,), jnp.int32)])
    ca_p = jnp.concatenate([ca, jnp.zeros((Tp - T,), jnp.int32)])
    ca_px = jnp.concatenate([ca, jnp.full((Tp - T,), jnp.int32(1 << 20))])

    h_pad = _sc_gather(atom_table, an_pad)
    h = h_pad[:N]
    patom = jnp.concatenate([h, jnp.pad(pos, ((0, 0), (0, 125)))], axis=1)
    gsrc = _sc_gather(patom, src)
    gdst = _sc_gather(patom, dst)

    m, scale_h, scale_out, vb, x = _edge_stage(
        gsrc, gdst, W_edge, b_edge, W_rbf_h, W_rbf_out, W_cbf_r, W_down[0])

    va = _sc_gather(vb, ba_p)
    vc = _sc_gather(vb, ca_p)
    basis = _basis_stage(va, vc, W_cbf_s)

    ms = [m]
    for b in range(3):
        xg = _sc_gather(x, ba_p)
        t = _t_stage(xg, basis, W_b2t[b], W_bil[b])
        aggp = _sc_tri_dense(t, ca_px, E)
        m, msh = _m1_stage(aggp, m, scale_h, W_up[b], W_res[b])
        amsgp = _sc_segsum([msh], dst, N)
        h, hsN, hdN = _h_stage(amsgp[0], h, W_atom[b], W_e2[b])
        ghs = _sc_gather_small(hsN, src)
        ghd = _sc_gather_small(hdN, dst)
        if b < 2:
            m, x = _m2_stage(m, ghs, ghd, W_e2[b], W_down[b + 1])
        else:
            m = _m2_stage(m, ghs, ghd, W_e2[b])
        ms.append(m)

    xm0, xm1, xm2, xm3, fV = _out_stage(ms, scale_out, vb, W_F)
    xap = _sc_segsum([xm0, xm1, xm2, xm3], dst, N)
    Eat = _energy_stage(xap, W_out1, W_out2)
    fvp = _sc_segsum([fV], dst, N)
    F = (fvp[0, 0] + fvp[0, 1])[:, :3]
    energy = jnp.sum(Eat).reshape(1)
    return jnp.concatenate([F.reshape(-1), energy])
